# baseline (speedup 1.0000x reference)
import functools

import jax
import jax.numpy as jnp
from jax import lax
from jax.experimental import pallas as pl
from jax.experimental.pallas import tpu as pltpu
from jax.experimental.pallas import tpu_sc as plsc

B, T = 4, 8192
V, D = 8192, 1024
B_TOTAL = B * T
TC_R = 256


def _tc_gather(idx_flat, pe, n_rows):
    grid = (n_rows // TC_R,)

    def body(idx_ref, pe_hbm, out_ref, sem):
        i = pl.program_id(0)
        base = i * TC_R

        def issue(r, c):
            row = idx_ref[base + r]
            pltpu.make_async_copy(pe_hbm.at[row], out_ref.at[r], sem).start()
            return c

        lax.fori_loop(0, TC_R, issue, 0, unroll=8)
        pltpu.make_async_copy(pe_hbm.at[pl.ds(0, TC_R)], out_ref, sem).wait()

    return pl.pallas_call(
        body,
        grid_spec=pltpu.PrefetchScalarGridSpec(
            num_scalar_prefetch=1,
            grid=grid,
            in_specs=[pl.BlockSpec(memory_space=pl.ANY)],
            out_specs=pl.BlockSpec((TC_R, D), lambda i, idx_ref: (i, 0)),
            scratch_shapes=[pltpu.SemaphoreType.DMA],
        ),
        out_shape=jax.ShapeDtypeStruct((n_rows, D), jnp.float32),
    )(idx_flat, pe)


@jax.jit
def kernel(t_indices, pe):
    idx = t_indices.astype(jnp.int32).reshape(B_TOTAL)
    out = _tc_gather(idx, pe, B_TOTAL)
    return out.reshape(B, T, D)


# speedup vs baseline: 1.3999x; 1.3999x over previous
import functools

import jax
import jax.numpy as jnp
from jax import lax
from jax.experimental import pallas as pl
from jax.experimental.pallas import tpu as pltpu
from jax.experimental.pallas import tpu_sc as plsc

B, T = 4, 8192
V, D = 8192, 1024
B_TOTAL = B * T

SC_ROWS = 24576
TC_ROWS = B_TOTAL - SC_ROWS

NC, NS = 2, 16
NW = NC * NS
B_PER_W = SC_ROWS // NW
CHUNK = 16
N_CHUNKS = B_PER_W // CHUNK
NBUF = 4

TC_R = 256


def _sc_gather(idx_sc, pe):
    mesh = plsc.VectorSubcoreMesh(core_axis_name="c", subcore_axis_name="s")

    @functools.partial(
        pl.kernel,
        mesh=mesh,
        out_type=jax.ShapeDtypeStruct((SC_ROWS, D), jnp.float32),
        scratch_types=[
            pltpu.VMEM((B_PER_W,), jnp.int32),
            [pltpu.VMEM((CHUNK, D), jnp.float32)] * NBUF,
            [pltpu.SemaphoreType.DMA] * NBUF,
            [pltpu.SemaphoreType.DMA] * NBUF,
        ],
    )
    def k(idx_hbm, table_hbm, out_hbm, idx_v, bufs, gsems, wsems):
        wid = lax.axis_index("s") * NC + lax.axis_index("c")
        base = wid * B_PER_W
        pltpu.sync_copy(idx_hbm.at[pl.ds(base, B_PER_W)], idx_v)

        def gather_start(c, b):
            pltpu.async_copy(
                table_hbm.at[idx_v.at[pl.ds(c * CHUNK, CHUNK)]], bufs[b], gsems[b]
            )

        def gather_wait(b):
            pltpu.make_async_copy(
                table_hbm.at[idx_v.at[pl.ds(0, CHUNK)]], bufs[b], gsems[b]
            ).wait()

        def write_start(c, b):
            pltpu.async_copy(
                bufs[b], out_hbm.at[pl.ds(base + c * CHUNK, CHUNK)], wsems[b]
            )

        def write_wait(b):
            pltpu.make_async_copy(
                bufs[b], out_hbm.at[pl.ds(base, CHUNK)], wsems[b]
            ).wait()

        for b in range(NBUF):
            gather_start(b, b)

        @pl.loop(0, N_CHUNKS - NBUF, step=NBUF)
        def group(g):
            for b in range(NBUF):
                gather_wait(b)
                write_start(g + b, b)
            for b in range(NBUF):
                write_wait(b)
                gather_start(g + NBUF + b, b)

        for b in range(NBUF):
            gather_wait(b)
            write_start(N_CHUNKS - NBUF + b, b)
        for b in range(NBUF):
            write_wait(b)

    return k(idx_sc, pe)


def _tc_gather(idx_tc, pe):
    def body(idx_ref, pe_hbm, out_ref, sem):
        i = pl.program_id(0)
        base = i * TC_R

        def issue(r, c):
            row = idx_ref[base + r]
            pltpu.make_async_copy(pe_hbm.at[row], out_ref.at[r], sem).start()
            return c

        lax.fori_loop(0, TC_R, issue, 0, unroll=8)
        pltpu.make_async_copy(pe_hbm.at[pl.ds(0, TC_R)], out_ref, sem).wait()

    return pl.pallas_call(
        body,
        grid_spec=pltpu.PrefetchScalarGridSpec(
            num_scalar_prefetch=1,
            grid=(TC_ROWS // TC_R,),
            in_specs=[pl.BlockSpec(memory_space=pl.ANY)],
            out_specs=pl.BlockSpec((TC_R, D), lambda i, idx_ref: (i, 0)),
            scratch_shapes=[pltpu.SemaphoreType.DMA],
        ),
        out_shape=jax.ShapeDtypeStruct((TC_ROWS, D), jnp.float32),
    )(idx_tc, pe)


@jax.jit
def kernel(t_indices, pe):
    idx = t_indices.astype(jnp.int32).reshape(B_TOTAL)
    sc_out = _sc_gather(idx[:SC_ROWS], pe)
    tc_out = _tc_gather(idx[SC_ROWS:], pe)
    out = jnp.concatenate([sc_out, tc_out], axis=0)
    return out.reshape(B, T, D)


# speedup vs baseline: 2.5309x; 1.8079x over previous
import functools

import jax
import jax.numpy as jnp
from jax import lax
from jax.experimental import pallas as pl
from jax.experimental.pallas import tpu as pltpu
from jax.experimental.pallas import tpu_sc as plsc

B, T = 4, 8192
V, D = 8192, 1024
B_TOTAL = B * T
NC, NS = 2, 16
NW = NC * NS
B_PER_W = B_TOTAL // NW
W_PER_ROW = T // B_PER_W
CHUNK = 16
N_CHUNKS = B_PER_W // CHUNK
NBUF = 4


def _sc_gather(t_indices, pe):
    mesh = plsc.VectorSubcoreMesh(core_axis_name="c", subcore_axis_name="s")

    @functools.partial(
        pl.kernel,
        mesh=mesh,
        out_type=jax.ShapeDtypeStruct((B_TOTAL, D), jnp.float32),
        scratch_types=[
            pltpu.VMEM((B_PER_W,), jnp.int32),
            [pltpu.VMEM((CHUNK, D), jnp.float32)] * NBUF,
            [pltpu.SemaphoreType.DMA] * NBUF,
            [pltpu.SemaphoreType.DMA] * NBUF,
        ],
    )
    def k(idx_hbm, table_hbm, out_hbm, idx_v, bufs, gsems, wsems):
        wid = lax.axis_index("s") * NC + lax.axis_index("c")
        base = wid * B_PER_W
        pltpu.sync_copy(
            idx_hbm.at[wid // W_PER_ROW, pl.ds((wid % W_PER_ROW) * B_PER_W, B_PER_W)],
            idx_v,
        )

        def gather_start(c, b):
            pltpu.async_copy(
                table_hbm.at[idx_v.at[pl.ds(c * CHUNK, CHUNK)]], bufs[b], gsems[b]
            )

        def gather_wait(b):
            pltpu.make_async_copy(
                table_hbm.at[idx_v.at[pl.ds(0, CHUNK)]], bufs[b], gsems[b]
            ).wait()

        def write_start(c, b):
            pltpu.async_copy(
                bufs[b], out_hbm.at[pl.ds(base + c * CHUNK, CHUNK)], wsems[b]
            )

        def write_wait(b):
            pltpu.make_async_copy(
                bufs[b], out_hbm.at[pl.ds(base, CHUNK)], wsems[b]
            ).wait()

        for b in range(NBUF):
            gather_start(b, b)

        @pl.loop(0, N_CHUNKS - NBUF, step=NBUF)
        def group(g):
            for b in range(NBUF):
                gather_wait(b)
                write_start(g + b, b)
            for b in range(NBUF):
                write_wait(b)
                gather_start(g + NBUF + b, b)

        for b in range(NBUF):
            gather_wait(b)
            write_start(N_CHUNKS - NBUF + b, b)
        for b in range(NBUF):
            write_wait(b)

    return k(t_indices, pe)


@jax.jit
def kernel(t_indices, pe):
    out = _sc_gather(t_indices.astype(jnp.int32), pe)
    return out.reshape(B, T, D)


# speedup vs baseline: 2.5731x; 1.0167x over previous
import functools

import jax
import jax.numpy as jnp
from jax import lax
from jax.experimental import pallas as pl
from jax.experimental.pallas import tpu as pltpu
from jax.experimental.pallas import tpu_sc as plsc

B, T = 4, 8192
V, D = 8192, 1024
B_TOTAL = B * T
NC, NS = 2, 16
NW = NC * NS
B_PER_W = B_TOTAL // NW
W_PER_ROW = T // B_PER_W
CHUNK = 8
N_CHUNKS = B_PER_W // CHUNK
NBUF = 8


def _sc_gather(t_indices, pe):
    mesh = plsc.VectorSubcoreMesh(core_axis_name="c", subcore_axis_name="s")

    @functools.partial(
        pl.kernel,
        mesh=mesh,
        out_type=jax.ShapeDtypeStruct((B_TOTAL, D), jnp.float32),
        scratch_types=[
            pltpu.VMEM((B_PER_W,), jnp.int32),
            [pltpu.VMEM((CHUNK, D), jnp.float32)] * NBUF,
            [pltpu.SemaphoreType.DMA] * NBUF,
            [pltpu.SemaphoreType.DMA] * NBUF,
        ],
    )
    def k(idx_hbm, table_hbm, out_hbm, idx_v, bufs, gsems, wsems):
        wid = lax.axis_index("s") * NC + lax.axis_index("c")
        base = wid * B_PER_W
        pltpu.sync_copy(
            idx_hbm.at[wid // W_PER_ROW, pl.ds((wid % W_PER_ROW) * B_PER_W, B_PER_W)],
            idx_v,
        )

        def gather_start(c, b):
            pltpu.async_copy(
                table_hbm.at[idx_v.at[pl.ds(c * CHUNK, CHUNK)]], bufs[b], gsems[b]
            )

        def gather_wait(b):
            pltpu.make_async_copy(
                table_hbm.at[idx_v.at[pl.ds(0, CHUNK)]], bufs[b], gsems[b]
            ).wait()

        def write_start(c, b):
            pltpu.async_copy(
                bufs[b], out_hbm.at[pl.ds(base + c * CHUNK, CHUNK)], wsems[b]
            )

        def write_wait(b):
            pltpu.make_async_copy(
                bufs[b], out_hbm.at[pl.ds(base, CHUNK)], wsems[b]
            ).wait()

        for b in range(NBUF):
            gather_start(b, b)

        @pl.loop(0, N_CHUNKS - NBUF, step=NBUF)
        def group(g):
            for b in range(NBUF):
                gather_wait(b)
                write_start(g + b, b)
            for b in range(NBUF):
                write_wait(b)
                gather_start(g + NBUF + b, b)

        for b in range(NBUF):
            gather_wait(b)
            write_start(N_CHUNKS - NBUF + b, b)
        for b in range(NBUF):
            write_wait(b)

    return k(t_indices, pe)


@jax.jit
def kernel(t_indices, pe):
    out = _sc_gather(t_indices.astype(jnp.int32), pe)
    return out.reshape(B, T, D)
